# baseline (speedup 1.0000x reference)
import functools

import jax
import jax.numpy as jnp
from jax import lax
from jax.experimental import pallas as pl
from jax.experimental.pallas import tpu as pltpu
from jax.experimental.pallas import tpu_sc as plsc


def _routing_kernel(x_ref, gw_ref, slot_ref, w16_ref, be_ref, bu_ref,
                    xbf_ref, *,
                    n_experts, block_m, nb):
    x = x_ref[...]
    n_tok = x.shape[0]
    na = 2 * n_tok
    xbf_ref[...] = x.astype(jnp.bfloat16)
    logits = jax.lax.dot_general(
        gw_ref[...], x, (((1,), (1,)), ((), ())),
        preferred_element_type=jnp.float32)
    scores = jax.nn.softmax(logits, axis=0)
    rows = jax.lax.broadcasted_iota(jnp.int32, scores.shape, 0)
    m1 = jnp.max(scores, axis=0, keepdims=True)
    i1 = jnp.argmax(scores, axis=0).astype(jnp.int32).reshape(1, n_tok)
    masked = jnp.where(rows == i1, -jnp.inf, scores)
    m2 = jnp.max(masked, axis=0, keepdims=True)
    i2 = jnp.argmax(masked, axis=0).astype(jnp.int32).reshape(1, n_tok)

    ea = jnp.concatenate([i1, i2], axis=1)
    wa = jnp.concatenate([m1, m2], axis=1)
    onehot = (jax.lax.broadcasted_iota(jnp.int32, (n_experts, na), 0)
              == ea).astype(jnp.int32)

    csum = onehot
    sh = 1
    while sh < na:
        shifted = jnp.concatenate(
            [jnp.zeros((n_experts, sh), jnp.int32), csum[:, :na - sh]], axis=1)
        csum = csum + shifted
        sh *= 2
    excl = csum - onehot
    counts = csum[:, na - 1:na]
    aligned = ((counts + block_m - 1) // block_m) * block_m
    s = aligned
    sh = 1
    while sh < n_experts:
        shifted = jnp.concatenate(
            [jnp.zeros((sh, 1), jnp.int32), s[:n_experts - sh]], axis=0)
        s = shifted + s
        sh *= 2
    starts = s - aligned
    ends = starts + aligned

    slot = jnp.sum((excl + starts) * onehot, axis=0).reshape(1, na)
    slot_ref[...] = slot.reshape(na)
    w16_ref[...] = jnp.transpose(wa) * jnp.ones((1, 128), jnp.float32)
    bidx = jax.lax.broadcasted_iota(jnp.int32, (1, nb), 1) * block_m
    be = jnp.sum((bidx >= ends).astype(jnp.int32), axis=0, keepdims=True)
    be_ref[...] = jnp.minimum(be, n_experts - 1)
    total_used = jnp.max(ends, axis=0, keepdims=True)
    bu_ref[...] = (bidx < total_used).astype(jnp.int32)


_SC_MESH = dict(core_axis_name="c", subcore_axis_name="s")
_NC, _NS = 2, 16
_NW = _NC * _NS


def _dispatch_sc(x, w16, slot, p_rows):
    n_tok, hid = x.shape
    na = slot.shape[0]
    bpw = na // _NW
    ch = 16
    nch = bpw // ch

    @functools.partial(
        pl.kernel,
        out_type=[jax.ShapeDtypeStruct((p_rows, hid), jnp.float32),
                  jax.ShapeDtypeStruct((p_rows, 128), jnp.float32)],
        mesh=plsc.VectorSubcoreMesh(**_SC_MESH),
        scratch_types=[pltpu.VMEM((2, ch, hid), jnp.float32),
                       pltpu.VMEM((2, ch, 128), jnp.float32),
                       pltpu.VMEM((2, ch), jnp.int32),
                       pltpu.SemaphoreType.DMA,
                       pltpu.SemaphoreType.DMA,
                       pltpu.SemaphoreType.DMA,
                       pltpu.SemaphoreType.DMA],
    )
    def disp(x_hbm, w_hbm, slot_hbm, xs_hbm, ws_hbm, rows_v, wv, idx_v,
             ld0, ld1, st0, st1):
        wid = lax.axis_index("s") * _NC + lax.axis_index("c")
        base = wid * bpw
        lds = (ld0, ld1)
        sts = (st0, st1)

        def issue_loads(c):
            off = base + c * ch
            k = c % 2
            return (
                pltpu.async_copy(slot_hbm.at[pl.ds(off, ch)],
                                 idx_v.at[k], lds[k]),
                pltpu.async_copy(x_hbm.at[pl.ds(lax.rem(off, n_tok), ch)],
                                 rows_v.at[k], lds[k]),
                pltpu.async_copy(w_hbm.at[pl.ds(off, ch)], wv.at[k], lds[k]),
            )

        loads = {0: issue_loads(0), 1: issue_loads(1)}
        scat = {}
        for c in range(nch):
            k = c % 2
            for h in loads.pop(c):
                h.wait()
            scat[c] = (
                pltpu.async_copy(rows_v.at[k], xs_hbm.at[idx_v.at[k]], sts[k]),
                pltpu.async_copy(wv.at[k], ws_hbm.at[idx_v.at[k]], sts[k]),
            )
            if c + 2 < nch:
                for h in scat.pop(c):
                    h.wait()
                loads[c + 2] = issue_loads(c + 2)
        for c in sorted(scat):
            for h in scat[c]:
                h.wait()

    return disp(x, w16, slot)


def _gather_sc(ys, slot):
    p_rows, hid = ys.shape
    na = slot.shape[0]
    bpw = na // _NW
    ch = 16
    nch = bpw // ch

    @functools.partial(
        pl.kernel,
        out_type=jax.ShapeDtypeStruct((na, hid), jnp.float32),
        mesh=plsc.VectorSubcoreMesh(**_SC_MESH),
        scratch_types=[pltpu.VMEM((2, ch, hid), jnp.float32),
                       pltpu.VMEM((nch, ch), jnp.int32),
                       pltpu.SemaphoreType.DMA,
                       pltpu.SemaphoreType.DMA,
                       pltpu.SemaphoreType.DMA,
                       pltpu.SemaphoreType.DMA],
    )
    def gath(ys_hbm, slot_hbm, g_hbm, rows_v, idx_v, g0, g1, s0, s1):
        wid = lax.axis_index("s") * _NC + lax.axis_index("c")
        base = wid * bpw
        gts = (g0, g1)
        sts = (s0, s1)
        ih = [pltpu.async_copy(slot_hbm.at[pl.ds(base + c * ch, ch)],
                               idx_v.at[c], g0) for c in range(nch)]
        for h in ih:
            h.wait()

        def issue_gather(c):
            k = c % 2
            return pltpu.async_copy(ys_hbm.at[idx_v.at[c]], rows_v.at[k],
                                    gts[k])

        fet = {0: issue_gather(0), 1: issue_gather(1)}
        sto = {}
        for c in range(nch):
            k = c % 2
            fet.pop(c).wait()
            sto[c] = pltpu.async_copy(rows_v.at[k],
                                      g_hbm.at[pl.ds(base + c * ch, ch)],
                                      sts[k])
            if c + 2 < nch:
                sto.pop(c).wait()
                fet[c + 2] = issue_gather(c + 2)
        for c in sorted(sto):
            sto[c].wait()

    return gath(ys, slot)


def _e1_kernel(be_ref, bu_ref, xs_ref, gp_ref, up_ref, h_ref):
    b = pl.program_id(0)

    @pl.when(bu_ref[b] != 0)
    def _():
        xb = xs_ref[...].astype(jnp.bfloat16)
        wg = gp_ref[0].astype(jnp.bfloat16)
        wu = up_ref[0].astype(jnp.bfloat16)
        xg = jax.lax.dot_general(xb, wg, (((1,), (1,)), ((), ())),
                                 preferred_element_type=jnp.float32)
        xu = jax.lax.dot_general(xb, wu, (((1,), (1,)), ((), ())),
                                 preferred_element_type=jnp.float32)
        h_ref[...] = (xg * jax.nn.sigmoid(xg) * xu).astype(jnp.bfloat16)


def _e2_kernel(be_ref, bu_ref, h_ref, dp_ref, ws_ref, ys_ref):
    b = pl.program_id(0)

    @pl.when(bu_ref[b] != 0)
    def _():
        h = h_ref[...]
        wd = dp_ref[0].astype(jnp.bfloat16)
        y = jax.lax.dot_general(h, wd, (((1,), (1,)), ((), ())),
                                preferred_element_type=jnp.float32)
        ys_ref[...] = y * ws_ref[:, :1]


def _s1_kernel(xbf_ref, sg_ref, su_ref, h_ref):
    xb = xbf_ref[...]
    wg = sg_ref[...].astype(jnp.bfloat16)
    wu = su_ref[...].astype(jnp.bfloat16)
    xg = jax.lax.dot_general(xb, wg, (((1,), (1,)), ((), ())),
                             preferred_element_type=jnp.float32)
    xu = jax.lax.dot_general(xb, wu, (((1,), (1,)), ((), ())),
                             preferred_element_type=jnp.float32)
    h_ref[...] = (xg * jax.nn.sigmoid(xg) * xu).astype(jnp.bfloat16)


def _s2_kernel(h_ref, sd_ref, g1_ref, g2_ref, out_ref):
    j = pl.program_id(1)
    wd = sd_ref[...].astype(jnp.bfloat16)
    acc = jax.lax.dot_general(h_ref[...], wd, (((1,), (1,)), ((), ())),
                              preferred_element_type=jnp.float32)

    @pl.when(j == 0)
    def _():
        out_ref[...] = g1_ref[...] + g2_ref[...] + acc

    @pl.when(j != 0)
    def _():
        out_ref[...] += acc


def kernel(hidden_states, gate_w, gate_proj_w, up_proj_w, down_proj_w,
           shared_gate_w, shared_up_w, shared_down_w):
    orig_shape = hidden_states.shape
    hid = orig_shape[-1]
    x = hidden_states.reshape(-1, hid)
    n_tok = x.shape[0]
    n_experts, moe_inter, _ = gate_proj_w.shape
    shared_inter = shared_gate_w.shape[0]
    na = 2 * n_tok

    block_m = min(128, n_tok)
    nb = na // block_m + n_experts
    p_rows = nb * block_m

    slot, w16, be_row, bu_row, xbf = pl.pallas_call(
        functools.partial(_routing_kernel, n_experts=n_experts,
                          block_m=block_m, nb=nb),
        out_shape=(
            jax.ShapeDtypeStruct((na,), jnp.int32),
            jax.ShapeDtypeStruct((na, 128), jnp.float32),
            jax.ShapeDtypeStruct((1, nb), jnp.int32),
            jax.ShapeDtypeStruct((1, nb), jnp.int32),
            jax.ShapeDtypeStruct((n_tok, hid), jnp.bfloat16),
        ),
    )(x, gate_w)
    be = be_row.reshape(nb)
    bu = bu_row.reshape(nb)

    xs, ws = _dispatch_sc(x, w16, slot, p_rows)

    tb = min(512, n_tok)
    ntb = n_tok // tb
    sjs = min(1024, shared_inter)
    njs = shared_inter // sjs
    h_sh = pl.pallas_call(
        _s1_kernel,
        grid=(njs, ntb),
        in_specs=[
            pl.BlockSpec((tb, hid), lambda j, t: (t, 0)),
            pl.BlockSpec((sjs, hid), lambda j, t: (j, 0)),
            pl.BlockSpec((sjs, hid), lambda j, t: (j, 0)),
        ],
        out_specs=pl.BlockSpec((tb, sjs), lambda j, t: (t, j)),
        out_shape=jax.ShapeDtypeStruct((n_tok, shared_inter), jnp.bfloat16),
    )(xbf, shared_gate_w, shared_up_w)

    h_moe = pl.pallas_call(
        _e1_kernel,
        grid_spec=pltpu.PrefetchScalarGridSpec(
            num_scalar_prefetch=2,
            grid=(nb,),
            in_specs=[
                pl.BlockSpec((block_m, hid), lambda b, be, bu: (b, 0)),
                pl.BlockSpec((1, moe_inter, hid),
                             lambda b, be, bu: (be[b], 0, 0)),
                pl.BlockSpec((1, moe_inter, hid),
                             lambda b, be, bu: (be[b], 0, 0)),
            ],
            out_specs=pl.BlockSpec((block_m, moe_inter),
                                   lambda b, be, bu: (b, 0)),
        ),
        out_shape=jax.ShapeDtypeStruct((p_rows, moe_inter), jnp.bfloat16),
    )(be, bu, xs, gate_proj_w, up_proj_w)

    ys = pl.pallas_call(
        _e2_kernel,
        grid_spec=pltpu.PrefetchScalarGridSpec(
            num_scalar_prefetch=2,
            grid=(nb,),
            in_specs=[
                pl.BlockSpec((block_m, moe_inter), lambda b, be, bu: (b, 0)),
                pl.BlockSpec((1, hid, moe_inter),
                             lambda b, be, bu: (be[b], 0, 0)),
                pl.BlockSpec((block_m, 128), lambda b, be, bu: (b, 0)),
            ],
            out_specs=pl.BlockSpec((block_m, hid), lambda b, be, bu: (b, 0)),
        ),
        out_shape=jax.ShapeDtypeStruct((p_rows, hid), jnp.float32),
    )(be, bu, h_moe, down_proj_w, ws)

    g = _gather_sc(ys, slot)

    sjk = min(1024, shared_inter)
    njk = shared_inter // sjk
    y = pl.pallas_call(
        _s2_kernel,
        grid=(ntb, njk),
        in_specs=[
            pl.BlockSpec((tb, sjk), lambda t, j: (t, j)),
            pl.BlockSpec((hid, sjk), lambda t, j: (0, j)),
            pl.BlockSpec((tb, hid), lambda t, j: (t, 0)),
            pl.BlockSpec((tb, hid), lambda t, j, _ntb=ntb: (t + _ntb, 0)),
        ],
        out_specs=pl.BlockSpec((tb, hid), lambda t, j: (t, 0)),
        out_shape=jax.ShapeDtypeStruct((n_tok, hid), jnp.float32),
    )(h_sh, shared_down_w, g, g)

    return y.reshape(orig_shape)


# speedup vs baseline: 1.2452x; 1.2452x over previous
import functools

import jax
import jax.numpy as jnp
from jax import lax
from jax.experimental import pallas as pl
from jax.experimental.pallas import tpu as pltpu
from jax.experimental.pallas import tpu_sc as plsc


def _routing_kernel(x_ref, gw_ref, slot_ref, w16_ref, be_ref, bu_ref,
                    xbf_ref, *,
                    n_experts, block_m, nb):
    x = x_ref[...]
    n_tok = x.shape[0]
    na = 2 * n_tok
    xbf_ref[...] = x.astype(jnp.bfloat16)
    logits = jax.lax.dot_general(
        gw_ref[...], x, (((1,), (1,)), ((), ())),
        preferred_element_type=jnp.float32)
    scores = jax.nn.softmax(logits, axis=0)
    rows = jax.lax.broadcasted_iota(jnp.int32, scores.shape, 0)
    m1 = jnp.max(scores, axis=0, keepdims=True)
    i1 = jnp.argmax(scores, axis=0).astype(jnp.int32).reshape(1, n_tok)
    masked = jnp.where(rows == i1, -jnp.inf, scores)
    m2 = jnp.max(masked, axis=0, keepdims=True)
    i2 = jnp.argmax(masked, axis=0).astype(jnp.int32).reshape(1, n_tok)

    ea = jnp.concatenate([i1, i2], axis=1)
    wa = jnp.concatenate([m1, m2], axis=1)
    onehot = (jax.lax.broadcasted_iota(jnp.int32, (n_experts, na), 0)
              == ea).astype(jnp.int32)

    csum = onehot
    sh = 1
    while sh < na:
        shifted = jnp.concatenate(
            [jnp.zeros((n_experts, sh), jnp.int32), csum[:, :na - sh]], axis=1)
        csum = csum + shifted
        sh *= 2
    excl = csum - onehot
    counts = csum[:, na - 1:na]
    aligned = ((counts + block_m - 1) // block_m) * block_m
    s = aligned
    sh = 1
    while sh < n_experts:
        shifted = jnp.concatenate(
            [jnp.zeros((sh, 1), jnp.int32), s[:n_experts - sh]], axis=0)
        s = shifted + s
        sh *= 2
    starts = s - aligned
    ends = starts + aligned

    slot = jnp.sum((excl + starts) * onehot, axis=0).reshape(1, na)
    slot_ref[...] = slot.reshape(na)
    w16_ref[...] = jnp.transpose(wa) * jnp.ones((1, 128), jnp.float32)
    bidx = jax.lax.broadcasted_iota(jnp.int32, (1, nb), 1) * block_m
    be = jnp.sum((bidx >= ends).astype(jnp.int32), axis=0, keepdims=True)
    be_ref[...] = jnp.minimum(be, n_experts - 1)
    total_used = jnp.max(ends, axis=0, keepdims=True)
    bu_ref[...] = (bidx < total_used).astype(jnp.int32)


_SC_MESH = dict(core_axis_name="c", subcore_axis_name="s")
_NC, _NS = 2, 16
_NW = _NC * _NS


def _dispatch_sc(x, w16, slot, p_rows):
    n_tok, hid = x.shape
    na = slot.shape[0]
    bpw = na // _NW
    ch = 16
    nch = bpw // ch

    @functools.partial(
        pl.kernel,
        out_type=[jax.ShapeDtypeStruct((p_rows, hid), jnp.float32),
                  jax.ShapeDtypeStruct((p_rows, 128), jnp.float32)],
        mesh=plsc.VectorSubcoreMesh(**_SC_MESH),
        scratch_types=[pltpu.VMEM((2, ch, hid), jnp.float32),
                       pltpu.VMEM((2, ch, 128), jnp.float32),
                       pltpu.VMEM((2, ch), jnp.int32),
                       pltpu.SemaphoreType.DMA,
                       pltpu.SemaphoreType.DMA,
                       pltpu.SemaphoreType.DMA,
                       pltpu.SemaphoreType.DMA],
    )
    def disp(x_hbm, w_hbm, slot_hbm, xs_hbm, ws_hbm, rows_v, wv, idx_v,
             ld0, ld1, st0, st1):
        wid = lax.axis_index("s") * _NC + lax.axis_index("c")
        base = wid * bpw
        lds = (ld0, ld1)
        sts = (st0, st1)

        def issue_loads(c):
            off = base + c * ch
            k = c % 2
            return (
                pltpu.async_copy(slot_hbm.at[pl.ds(off, ch)],
                                 idx_v.at[k], lds[k]),
                pltpu.async_copy(x_hbm.at[pl.ds(lax.rem(off, n_tok), ch)],
                                 rows_v.at[k], lds[k]),
                pltpu.async_copy(w_hbm.at[pl.ds(off, ch)], wv.at[k], lds[k]),
            )

        loads = {0: issue_loads(0), 1: issue_loads(1)}
        scat = {}
        for c in range(nch):
            k = c % 2
            for h in loads.pop(c):
                h.wait()
            scat[c] = (
                pltpu.async_copy(rows_v.at[k], xs_hbm.at[idx_v.at[k]], sts[k]),
                pltpu.async_copy(wv.at[k], ws_hbm.at[idx_v.at[k]], sts[k]),
            )
            if c + 2 < nch:
                for h in scat.pop(c):
                    h.wait()
                loads[c + 2] = issue_loads(c + 2)
        for c in sorted(scat):
            for h in scat[c]:
                h.wait()

    return disp(x, w16, slot)


def _gather_sc(ys, slot):
    p_rows, hid = ys.shape
    na = slot.shape[0]
    bpw = na // _NW
    ch = 16
    nch = bpw // ch

    @functools.partial(
        pl.kernel,
        out_type=jax.ShapeDtypeStruct((na, hid), jnp.float32),
        mesh=plsc.VectorSubcoreMesh(**_SC_MESH),
        scratch_types=[pltpu.VMEM((2, ch, hid), jnp.float32),
                       pltpu.VMEM((nch, ch), jnp.int32),
                       pltpu.SemaphoreType.DMA,
                       pltpu.SemaphoreType.DMA,
                       pltpu.SemaphoreType.DMA,
                       pltpu.SemaphoreType.DMA],
    )
    def gath(ys_hbm, slot_hbm, g_hbm, rows_v, idx_v, g0, g1, s0, s1):
        wid = lax.axis_index("s") * _NC + lax.axis_index("c")
        base = wid * bpw
        gts = (g0, g1)
        sts = (s0, s1)
        ih = [pltpu.async_copy(slot_hbm.at[pl.ds(base + c * ch, ch)],
                               idx_v.at[c], g0) for c in range(nch)]
        for h in ih:
            h.wait()

        def issue_gather(c):
            k = c % 2
            return pltpu.async_copy(ys_hbm.at[idx_v.at[c]], rows_v.at[k],
                                    gts[k])

        fet = {0: issue_gather(0), 1: issue_gather(1)}
        sto = {}
        for c in range(nch):
            k = c % 2
            fet.pop(c).wait()
            sto[c] = pltpu.async_copy(rows_v.at[k],
                                      g_hbm.at[pl.ds(base + c * ch, ch)],
                                      sts[k])
            if c + 2 < nch:
                sto.pop(c).wait()
                fet[c + 2] = issue_gather(c + 2)
        for c in sorted(sto):
            sto[c].wait()

    return gath(ys, slot)


def _e1_kernel(be_ref, bu_ref, xs_ref, gp_ref, up_ref, h_ref):
    b = pl.program_id(0)

    @pl.when(bu_ref[b] != 0)
    def _():
        xb = xs_ref[...].astype(jnp.bfloat16)
        wg = gp_ref[0].astype(jnp.bfloat16)
        wu = up_ref[0].astype(jnp.bfloat16)
        xg = jax.lax.dot_general(xb, wg, (((1,), (1,)), ((), ())),
                                 preferred_element_type=jnp.float32)
        xu = jax.lax.dot_general(xb, wu, (((1,), (1,)), ((), ())),
                                 preferred_element_type=jnp.float32)
        h_ref[...] = (xg * jax.nn.sigmoid(xg) * xu).astype(jnp.bfloat16)


def _e2_kernel(be_ref, bu_ref, h_ref, dp_ref, ws_ref, ys_ref):
    b = pl.program_id(0)

    @pl.when(bu_ref[b] != 0)
    def _():
        h = h_ref[...]
        wd = dp_ref[0].astype(jnp.bfloat16)
        y = jax.lax.dot_general(h, wd, (((1,), (1,)), ((), ())),
                                preferred_element_type=jnp.float32)
        ys_ref[...] = y * ws_ref[:, :1]


def _s1_kernel(xbf_ref, sg_ref, su_ref, h_ref):
    xb = xbf_ref[...]
    wg = sg_ref[...].astype(jnp.bfloat16)
    wu = su_ref[...].astype(jnp.bfloat16)
    xg = jax.lax.dot_general(xb, wg, (((1,), (1,)), ((), ())),
                             preferred_element_type=jnp.float32)
    xu = jax.lax.dot_general(xb, wu, (((1,), (1,)), ((), ())),
                             preferred_element_type=jnp.float32)
    h_ref[...] = (xg * jax.nn.sigmoid(xg) * xu).astype(jnp.bfloat16)


def _s2_kernel(h_ref, sd_ref, g1_ref, g2_ref, out_ref):
    j = pl.program_id(1)
    wd = sd_ref[...].astype(jnp.bfloat16)
    acc = jax.lax.dot_general(h_ref[...], wd, (((1,), (1,)), ((), ())),
                              preferred_element_type=jnp.float32)

    @pl.when(j == 0)
    def _():
        out_ref[...] = g1_ref[...] + g2_ref[...] + acc

    @pl.when(j != 0)
    def _():
        out_ref[...] += acc


def kernel(hidden_states, gate_w, gate_proj_w, up_proj_w, down_proj_w,
           shared_gate_w, shared_up_w, shared_down_w):
    orig_shape = hidden_states.shape
    hid = orig_shape[-1]
    x = hidden_states.reshape(-1, hid)
    n_tok = x.shape[0]
    n_experts, moe_inter, _ = gate_proj_w.shape
    shared_inter = shared_gate_w.shape[0]
    na = 2 * n_tok

    block_m = min(256, n_tok)
    nb = na // block_m + n_experts
    p_rows = nb * block_m

    slot, w16, be_row, bu_row, xbf = pl.pallas_call(
        functools.partial(_routing_kernel, n_experts=n_experts,
                          block_m=block_m, nb=nb),
        out_shape=(
            jax.ShapeDtypeStruct((na,), jnp.int32),
            jax.ShapeDtypeStruct((na, 128), jnp.float32),
            jax.ShapeDtypeStruct((1, nb), jnp.int32),
            jax.ShapeDtypeStruct((1, nb), jnp.int32),
            jax.ShapeDtypeStruct((n_tok, hid), jnp.bfloat16),
        ),
    )(x, gate_w)
    be = be_row.reshape(nb)
    bu = bu_row.reshape(nb)

    xs, ws = _dispatch_sc(x, w16, slot, p_rows)

    tb = min(512, n_tok)
    ntb = n_tok // tb
    sjs = min(1024, shared_inter)
    njs = shared_inter // sjs
    h_sh = pl.pallas_call(
        _s1_kernel,
        grid=(njs, ntb),
        in_specs=[
            pl.BlockSpec((tb, hid), lambda j, t: (t, 0)),
            pl.BlockSpec((sjs, hid), lambda j, t: (j, 0)),
            pl.BlockSpec((sjs, hid), lambda j, t: (j, 0)),
        ],
        out_specs=pl.BlockSpec((tb, sjs), lambda j, t: (t, j)),
        out_shape=jax.ShapeDtypeStruct((n_tok, shared_inter), jnp.bfloat16),
    )(xbf, shared_gate_w, shared_up_w)

    h_moe = pl.pallas_call(
        _e1_kernel,
        grid_spec=pltpu.PrefetchScalarGridSpec(
            num_scalar_prefetch=2,
            grid=(nb,),
            in_specs=[
                pl.BlockSpec((block_m, hid), lambda b, be, bu: (b, 0)),
                pl.BlockSpec((1, moe_inter, hid),
                             lambda b, be, bu: (be[b], 0, 0)),
                pl.BlockSpec((1, moe_inter, hid),
                             lambda b, be, bu: (be[b], 0, 0)),
            ],
            out_specs=pl.BlockSpec((block_m, moe_inter),
                                   lambda b, be, bu: (b, 0)),
        ),
        out_shape=jax.ShapeDtypeStruct((p_rows, moe_inter), jnp.bfloat16),
    )(be, bu, xs, gate_proj_w, up_proj_w)

    ys = pl.pallas_call(
        _e2_kernel,
        grid_spec=pltpu.PrefetchScalarGridSpec(
            num_scalar_prefetch=2,
            grid=(nb,),
            in_specs=[
                pl.BlockSpec((block_m, moe_inter), lambda b, be, bu: (b, 0)),
                pl.BlockSpec((1, hid, moe_inter),
                             lambda b, be, bu: (be[b], 0, 0)),
                pl.BlockSpec((block_m, 128), lambda b, be, bu: (b, 0)),
            ],
            out_specs=pl.BlockSpec((block_m, hid), lambda b, be, bu: (b, 0)),
        ),
        out_shape=jax.ShapeDtypeStruct((p_rows, hid), jnp.float32),
    )(be, bu, h_moe, down_proj_w, ws)

    g = _gather_sc(ys, slot)

    sjk = min(1024, shared_inter)
    njk = shared_inter // sjk
    y = pl.pallas_call(
        _s2_kernel,
        grid=(ntb, njk),
        in_specs=[
            pl.BlockSpec((tb, sjk), lambda t, j: (t, j)),
            pl.BlockSpec((hid, sjk), lambda t, j: (0, j)),
            pl.BlockSpec((tb, hid), lambda t, j: (t, 0)),
            pl.BlockSpec((tb, hid), lambda t, j, _ntb=ntb: (t + _ntb, 0)),
        ],
        out_specs=pl.BlockSpec((tb, hid), lambda t, j: (t, 0)),
        out_shape=jax.ShapeDtypeStruct((n_tok, hid), jnp.float32),
    )(h_sh, shared_down_w, g, g)

    return y.reshape(orig_shape)
